# baseline (speedup 1.0000x reference)
import functools

import jax
import jax.numpy as jnp
from jax import lax
from jax.experimental import pallas as pl
from jax.experimental.pallas import tpu as pltpu
from jax.experimental.pallas import tpu_sc as plsc

NC = 2
NS = 16
CK = 128
D = 128


def _sc_aggregate(n_pad, ept):
  rpt = n_pad // NS
  mesh = plsc.VectorSubcoreMesh(
      core_axis_name="c", subcore_axis_name="s", num_cores=NC,
      num_subcores=NS)

  @functools.partial(
      pl.kernel, mesh=mesh,
      out_type=jax.ShapeDtypeStruct((NC, n_pad, D), jnp.float32),
      scratch_types=[
          pltpu.VMEM((ept, CK), jnp.int32),
          pltpu.VMEM((ept, CK), jnp.int32),
          pltpu.VMEM((CK, D), jnp.float32),
          pltpu.VMEM_SHARED((n_pad, D), jnp.float32),
          pltpu.SemaphoreType.DMA,
      ])
  def body(h_hbm, src_hbm, dst_hbm, zrow_hbm, agg_out, sidx, didx, rows,
           acc, sem):
    c = lax.axis_index("c")
    s = lax.axis_index("s")
    row0 = s * rpt

    pltpu.sync_copy(zrow_hbm, acc.at[pl.ds(row0, rpt)])
    base = (c * NS + s) * ept
    pltpu.sync_copy(src_hbm.at[pl.ds(base, ept)], sidx)
    pltpu.sync_copy(dst_hbm.at[pl.ds(base, ept)], didx)
    plsc.subcore_barrier()

    def step(j, carry):
      pltpu.async_copy(h_hbm.at[sidx.at[j]], rows, sem).wait()
      pltpu.sync_copy(rows, acc.at[didx.at[j]], add=True)
      return carry

    lax.fori_loop(0, ept, step, 0)
    plsc.subcore_barrier()

    pltpu.sync_copy(acc.at[pl.ds(row0, rpt)],
                    agg_out.at[c, pl.ds(row0, rpt)])

  return body


def _tc_dense(n, h_out, bn, relu):
  def body(h_ref, pa_ref, pb_ref, da_ref, db_ref, ws_ref, wn_ref, b_ref,
           o_ref):
    deg = da_ref[0, :, 0:1] + db_ref[0, :, 0:1]
    inv = 1.0 / jnp.maximum(deg, 1.0)
    agg = (pa_ref[0] + pb_ref[0]) * inv
    out = (
        jnp.dot(h_ref[...], ws_ref[...], preferred_element_type=jnp.float32)
        + jnp.dot(agg, wn_ref[...], preferred_element_type=jnp.float32)
        + b_ref[...])
    if relu:
      out = jnp.maximum(out, 0.0)
    o_ref[...] = out

  grid = (n // bn,)
  return pl.pallas_call(
      body,
      grid=grid,
      in_specs=[
          pl.BlockSpec((bn, D), lambda i: (i, 0)),
          pl.BlockSpec((1, bn, D), lambda i: (0, i, 0)),
          pl.BlockSpec((1, bn, D), lambda i: (1, i, 0)),
          pl.BlockSpec((1, bn, D), lambda i: (0, i, 0)),
          pl.BlockSpec((1, bn, D), lambda i: (1, i, 0)),
          pl.BlockSpec((D, h_out), lambda i: (0, 0)),
          pl.BlockSpec((D, h_out), lambda i: (0, 0)),
          pl.BlockSpec((1, h_out), lambda i: (0, 0)),
      ],
      out_specs=pl.BlockSpec((bn, h_out), lambda i: (i, 0)),
      out_shape=jax.ShapeDtypeStruct((n, h_out), jnp.float32),
  )


def kernel(x, edge_index, W_self0, W_neigh0, b0, W_self1, W_neigh1, b1,
           W_self2, W_neigh2, b2):
  n = x.shape[0]
  e = edge_index.shape[1]
  c_out = W_self2.shape[1]

  ept = -(-e // (NC * NS * CK * 8)) * 8
  e_pad = NC * NS * CK * ept
  n_pad = -(-(n + 1) // (NS * 8)) * (NS * 8)

  src = edge_index[0]
  dst = edge_index[1]
  pad = e_pad - e
  if pad:
    src = jnp.concatenate([src, jnp.zeros((pad,), jnp.int32)])
    dst = jnp.concatenate([dst, jnp.full((pad,), n, jnp.int32)])
  src2d = src.reshape(-1, CK)
  dst2d = dst.reshape(-1, CK)

  rpt = n_pad // NS
  zrow = jnp.zeros((rpt, D), jnp.float32)
  ones_tab = jnp.ones((8, D), jnp.float32)
  zsrc2d = jnp.zeros_like(src2d)

  agg_only = _sc_aggregate(n_pad, ept)
  bn = 400
  dense0 = _tc_dense(n, W_self0.shape[1], bn, relu=True)
  dense1 = _tc_dense(n, W_self1.shape[1], bn, relu=True)
  dense2 = _tc_dense(n, c_out, bn, relu=False)

  b0r = b0.reshape(1, -1)
  b1r = b1.reshape(1, -1)
  b2r = b2.reshape(1, -1)

  degp = agg_only(ones_tab, zsrc2d, dst2d, zrow)
  agg0 = agg_only(x, src2d, dst2d, zrow)
  h1 = dense0(x, agg0, agg0, degp, degp, W_self0, W_neigh0, b0r)
  agg1 = agg_only(h1, src2d, dst2d, zrow)
  h2 = dense1(h1, agg1, agg1, degp, degp, W_self1, W_neigh1, b1r)
  agg2 = agg_only(h2, src2d, dst2d, zrow)
  out = dense2(h2, agg2, agg2, degp, degp, W_self2, W_neigh2, b2r)
  return out


# speedup vs baseline: 5.8821x; 5.8821x over previous
import functools

import jax
import jax.numpy as jnp
from jax import lax
from jax.experimental import pallas as pl
from jax.experimental.pallas import tpu as pltpu
from jax.experimental.pallas import tpu_sc as plsc

NC = 2
NS = 16
CK = 128
D = 128


def _sc_aggregate(n_pad, ept):
  rpt = n_pad // NS
  mesh = plsc.VectorSubcoreMesh(
      core_axis_name="c", subcore_axis_name="s", num_cores=NC,
      num_subcores=NS)

  @functools.partial(
      pl.kernel, mesh=mesh,
      out_type=jax.ShapeDtypeStruct((NC, n_pad, D), jnp.float32),
      scratch_types=[
          pltpu.VMEM((ept, CK), jnp.int32),
          pltpu.VMEM((ept, CK), jnp.int32),
          pltpu.VMEM((CK, D), jnp.float32),
          pltpu.VMEM_SHARED((n_pad, D), jnp.float32),
          pltpu.SemaphoreType.DMA,
      ])
  def body(h_hbm, src_hbm, dst_hbm, zrow_hbm, agg_out, sidx, didx, rows,
           acc, sem):
    c = lax.axis_index("c")
    s = lax.axis_index("s")
    row0 = s * rpt

    pltpu.sync_copy(zrow_hbm, acc.at[pl.ds(row0, rpt)])
    base = (c * NS + s) * ept
    pltpu.sync_copy(src_hbm.at[pl.ds(base, ept)], sidx)
    pltpu.sync_copy(dst_hbm.at[pl.ds(base, ept)], didx)
    plsc.subcore_barrier()

    def step(j, carry):
      pltpu.async_copy(h_hbm.at[sidx.at[j]], rows, sem).wait()
      pltpu.sync_copy(rows, acc.at[didx.at[j]], add=True)
      return carry

    lax.fori_loop(0, ept, step, 0)
    plsc.subcore_barrier()

    pltpu.sync_copy(acc.at[pl.ds(row0, rpt)],
                    agg_out.at[c, pl.ds(row0, rpt)])

  return body


def _tc_dense(n, h_out, bn, relu):
  def body(h_ref, pa_ref, pb_ref, da_ref, db_ref, ws_ref, wn_ref, b_ref,
           o_ref):
    deg = da_ref[0, :, 0:1] + db_ref[0, :, 0:1]
    inv = 1.0 / jnp.maximum(deg, 1.0)
    agg = (pa_ref[0] + pb_ref[0]) * inv
    out = (
        jnp.dot(h_ref[...], ws_ref[...], preferred_element_type=jnp.float32)
        + jnp.dot(agg, wn_ref[...], preferred_element_type=jnp.float32)
        + b_ref[...])
    if relu:
      out = jnp.maximum(out, 0.0)
    o_ref[...] = out

  grid = (n // bn,)
  return pl.pallas_call(
      body,
      grid=grid,
      in_specs=[
          pl.BlockSpec((bn, D), lambda i: (i, 0)),
          pl.BlockSpec((1, bn, D), lambda i: (0, i, 0)),
          pl.BlockSpec((1, bn, D), lambda i: (1, i, 0)),
          pl.BlockSpec((1, bn, D), lambda i: (0, i, 0)),
          pl.BlockSpec((1, bn, D), lambda i: (1, i, 0)),
          pl.BlockSpec((D, h_out), lambda i: (0, 0)),
          pl.BlockSpec((D, h_out), lambda i: (0, 0)),
          pl.BlockSpec((1, h_out), lambda i: (0, 0)),
      ],
      out_specs=pl.BlockSpec((bn, h_out), lambda i: (i, 0)),
      out_shape=jax.ShapeDtypeStruct((n, h_out), jnp.float32),
  )


def kernel(x, edge_index, W_self0, W_neigh0, b0, W_self1, W_neigh1, b1,
           W_self2, W_neigh2, b2):
  n = x.shape[0]
  e = edge_index.shape[1]
  c_out = W_self2.shape[1]

  ept = -(-e // (NC * NS * CK * 8)) * 8
  e_pad = NC * NS * CK * ept
  n_pad = -(-(n + 1) // (NS * 8)) * (NS * 8)

  src = edge_index[0]
  dst = edge_index[1]
  pad = e_pad - e
  if pad:
    src = jnp.concatenate([src, jnp.zeros((pad,), jnp.int32)])
    dst = jnp.concatenate([dst, jnp.full((pad,), n, jnp.int32)])
  src2d = src.reshape(-1, CK)
  dst2d = dst.reshape(-1, CK)

  rpt = n_pad // NS
  zrow = jnp.zeros((rpt, D), jnp.float32)
  ones_tab = jnp.ones((n, D), jnp.float32)

  agg_only = _sc_aggregate(n_pad, ept)
  bn = 400
  dense0 = _tc_dense(n, W_self0.shape[1], bn, relu=True)
  dense1 = _tc_dense(n, W_self1.shape[1], bn, relu=True)
  dense2 = _tc_dense(n, c_out, bn, relu=False)

  b0r = b0.reshape(1, -1)
  b1r = b1.reshape(1, -1)
  b2r = b2.reshape(1, -1)

  degp = agg_only(ones_tab, src2d, dst2d, zrow)
  agg0 = agg_only(x, src2d, dst2d, zrow)
  h1 = dense0(x, agg0, agg0, degp, degp, W_self0, W_neigh0, b0r)
  agg1 = agg_only(h1, src2d, dst2d, zrow)
  h2 = dense1(h1, agg1, agg1, degp, degp, W_self1, W_neigh1, b1r)
  agg2 = agg_only(h2, src2d, dst2d, zrow)
  out = dense2(h2, agg2, agg2, degp, degp, W_self2, W_neigh2, b2r)
  return out


# speedup vs baseline: 7.5399x; 1.2818x over previous
import functools

import jax
import jax.numpy as jnp
from jax import lax
from jax.experimental import pallas as pl
from jax.experimental.pallas import tpu as pltpu
from jax.experimental.pallas import tpu_sc as plsc

NC = 2
NS = 16
CK = 128
D = 128


def _sc_aggregate(n_pad, ept):
  rpt = n_pad // NS
  mesh = plsc.VectorSubcoreMesh(
      core_axis_name="c", subcore_axis_name="s", num_cores=NC,
      num_subcores=NS)

  @functools.partial(
      pl.kernel, mesh=mesh,
      out_type=jax.ShapeDtypeStruct((NC, n_pad, D), jnp.float32),
      scratch_types=[
          pltpu.VMEM((ept, CK), jnp.int32),
          pltpu.VMEM((ept, CK), jnp.int32),
          pltpu.VMEM((CK, D), jnp.float32),
          pltpu.VMEM_SHARED((n_pad, D), jnp.float32),
          pltpu.SemaphoreType.DMA,
      ])
  def body(h_hbm, src_hbm, dst_hbm, zrow_hbm, agg_out, sidx, didx, rows,
           acc, sem):
    c = lax.axis_index("c")
    s = lax.axis_index("s")
    row0 = s * rpt

    pltpu.sync_copy(zrow_hbm, acc.at[pl.ds(row0, rpt)])
    base = (c * NS + s) * ept
    pltpu.sync_copy(src_hbm.at[pl.ds(base, ept)], sidx)
    pltpu.sync_copy(dst_hbm.at[pl.ds(base, ept)], didx)
    plsc.subcore_barrier()

    def step(j, carry):
      pltpu.async_copy(h_hbm.at[sidx.at[j]], rows, sem).wait()
      pltpu.sync_copy(rows, acc.at[didx.at[j]], add=True)
      return carry

    lax.fori_loop(0, ept, step, 0)
    plsc.subcore_barrier()

    pltpu.sync_copy(acc.at[pl.ds(row0, rpt)],
                    agg_out.at[c, pl.ds(row0, rpt)])

  return body


def _sc_degree(n_pad, ept):
  rpt = n_pad // NS
  mesh = plsc.VectorSubcoreMesh(
      core_axis_name="c", subcore_axis_name="s", num_cores=NC,
      num_subcores=NS)

  @functools.partial(
      pl.kernel, mesh=mesh,
      out_type=jax.ShapeDtypeStruct((NC, n_pad, D), jnp.float32),
      scratch_types=[
          pltpu.VMEM((ept, CK), jnp.int32),
          pltpu.VMEM((CK, D), jnp.float32),
          pltpu.VMEM_SHARED((n_pad, D), jnp.float32),
      ])
  def body(dst_hbm, zrow_hbm, ones_hbm, deg_out, didx, ones_v, dacc):
    c = lax.axis_index("c")
    s = lax.axis_index("s")
    row0 = s * rpt
    pltpu.sync_copy(zrow_hbm, dacc.at[pl.ds(row0, rpt)])
    base = (c * NS + s) * ept
    pltpu.sync_copy(dst_hbm.at[pl.ds(base, ept)], didx)
    pltpu.sync_copy(ones_hbm, ones_v)
    plsc.subcore_barrier()

    def step(j, carry):
      pltpu.sync_copy(ones_v, dacc.at[didx.at[j]], add=True)
      return carry

    lax.fori_loop(0, ept, step, 0)
    plsc.subcore_barrier()
    pltpu.sync_copy(dacc.at[pl.ds(row0, rpt)],
                    deg_out.at[c, pl.ds(row0, rpt)])

  return body


def _tc_dense(n, h_out, bn, relu):
  def body(h_ref, pa_ref, pb_ref, da_ref, db_ref, ws_ref, wn_ref, b_ref,
           o_ref):
    deg = da_ref[0, :, 0:1] + db_ref[0, :, 0:1]
    inv = 1.0 / jnp.maximum(deg, 1.0)
    agg = (pa_ref[0] + pb_ref[0]) * inv
    out = (
        jnp.dot(h_ref[...], ws_ref[...], preferred_element_type=jnp.float32)
        + jnp.dot(agg, wn_ref[...], preferred_element_type=jnp.float32)
        + b_ref[...])
    if relu:
      out = jnp.maximum(out, 0.0)
    o_ref[...] = out

  grid = (n // bn,)
  return pl.pallas_call(
      body,
      grid=grid,
      in_specs=[
          pl.BlockSpec((bn, D), lambda i: (i, 0)),
          pl.BlockSpec((1, bn, D), lambda i: (0, i, 0)),
          pl.BlockSpec((1, bn, D), lambda i: (1, i, 0)),
          pl.BlockSpec((1, bn, D), lambda i: (0, i, 0)),
          pl.BlockSpec((1, bn, D), lambda i: (1, i, 0)),
          pl.BlockSpec((D, h_out), lambda i: (0, 0)),
          pl.BlockSpec((D, h_out), lambda i: (0, 0)),
          pl.BlockSpec((1, h_out), lambda i: (0, 0)),
      ],
      out_specs=pl.BlockSpec((bn, h_out), lambda i: (i, 0)),
      out_shape=jax.ShapeDtypeStruct((n, h_out), jnp.float32),
  )


def kernel(x, edge_index, W_self0, W_neigh0, b0, W_self1, W_neigh1, b1,
           W_self2, W_neigh2, b2):
  n = x.shape[0]
  e = edge_index.shape[1]
  c_out = W_self2.shape[1]

  ept = -(-e // (NC * NS * CK * 8)) * 8
  e_pad = NC * NS * CK * ept
  n_pad = -(-(n + 1) // (NS * 8)) * (NS * 8)

  src = edge_index[0]
  dst = edge_index[1]
  pad = e_pad - e
  if pad:
    src = jnp.concatenate([src, jnp.zeros((pad,), jnp.int32)])
    dst = jnp.concatenate([dst, jnp.full((pad,), n, jnp.int32)])
  src2d = src.reshape(-1, CK)
  dst2d = dst.reshape(-1, CK)

  rpt = n_pad // NS
  zrow = jnp.zeros((rpt, D), jnp.float32)
  ones_rows = jnp.ones((CK, D), jnp.float32)

  agg_only = _sc_aggregate(n_pad, ept)
  deg_kernel = _sc_degree(n_pad, ept)
  bn = 400
  dense0 = _tc_dense(n, W_self0.shape[1], bn, relu=True)
  dense1 = _tc_dense(n, W_self1.shape[1], bn, relu=True)
  dense2 = _tc_dense(n, c_out, bn, relu=False)

  b0r = b0.reshape(1, -1)
  b1r = b1.reshape(1, -1)
  b2r = b2.reshape(1, -1)

  degp = deg_kernel(dst2d, zrow, ones_rows)
  agg0 = agg_only(x, src2d, dst2d, zrow)
  h1 = dense0(x, agg0, agg0, degp, degp, W_self0, W_neigh0, b0r)
  agg1 = agg_only(h1, src2d, dst2d, zrow)
  h2 = dense1(h1, agg1, agg1, degp, degp, W_self1, W_neigh1, b1r)
  agg2 = agg_only(h2, src2d, dst2d, zrow)
  out = dense2(h2, agg2, agg2, degp, degp, W_self2, W_neigh2, b2r)
  return out
